# baseline (speedup 1.0000x reference)
import functools

import jax
import jax.numpy as jnp
from jax import lax
from jax.experimental import pallas as pl
from jax.experimental.pallas import tpu as pltpu
from jax.experimental.pallas import tpu_sc as plsc

HIDDEN = 2048
HEADS = 16
T = 256
NW = 32


def _sc_row_gather(idx, src, n_out, chunk):
    H = src.shape[1]
    rows_per_w = n_out // NW
    nchunks = rows_per_w // chunk
    mesh = plsc.VectorSubcoreMesh(core_axis_name="c", subcore_axis_name="s")

    @functools.partial(
        pl.kernel,
        out_type=jax.ShapeDtypeStruct((n_out, H), src.dtype),
        mesh=mesh,
        scratch_types=[
            pltpu.VMEM((chunk,), jnp.int32),
            pltpu.VMEM((chunk, H), src.dtype),
            pltpu.SemaphoreType.DMA,
        ],
    )
    def gather_k(idx_hbm, src_hbm, out_hbm, idx_v, rows_v, sem):
        wid = lax.axis_index("s") * 2 + lax.axis_index("c")
        base = wid * rows_per_w
        for c in range(nchunks):
            off = base + c * chunk
            pltpu.sync_copy(idx_hbm.at[pl.ds(off, chunk)], idx_v)
            pltpu.async_copy(src_hbm.at[idx_v], rows_v, sem).wait()
            pltpu.sync_copy(rows_v, out_hbm.at[pl.ds(off, chunk)])

    return gather_k(idx, src)


def _mm_body(roles_ref, x_ref, w_ref, b_ref, out_ref):
    x = x_ref[...].astype(jnp.bfloat16)
    w = w_ref[0]
    acc = lax.dot_general(x, w, (((1,), (1,)), ((), ())),
                          preferred_element_type=jnp.float32)
    out_ref[...] = acc + b_ref[0]


def kernel(x, pair_roles, W_sub, b_sub, W_obj, b_obj, W_val, b_val):
    B, P, H = x.shape
    N = B * P
    NPAD = N + 3 * T
    NTILES = NPAD // T

    r32 = pair_roles.astype(jnp.int32)
    roles_flat = jnp.broadcast_to(r32[None, :], (B, P)).reshape(N)
    order = jnp.argsort(roles_flat, stable=True).astype(jnp.int32)
    sorted_roles = roles_flat[order]
    counts = jnp.bincount(roles_flat, length=3)
    seg_len = ((counts + T - 1) // T) * T
    seg_start = jnp.concatenate([jnp.zeros(1, jnp.int32),
                                 jnp.cumsum(seg_len)[:2].astype(jnp.int32)])
    cum_counts = jnp.concatenate([jnp.zeros(1, jnp.int32),
                                  jnp.cumsum(counts)[:2].astype(jnp.int32)])
    j = jnp.arange(N, dtype=jnp.int32)
    padded_pos = seg_start[sorted_roles] + (j - cum_counts[sorted_roles])
    gather_idx = jnp.zeros(NPAD, jnp.int32).at[padded_pos].set(order)
    unsort_idx = jnp.zeros(N, jnp.int32).at[order].set(padded_pos)
    tile_base = jnp.arange(NTILES, dtype=jnp.int32) * T
    tile_role = ((tile_base >= seg_start[1]).astype(jnp.int32)
                 + (tile_base >= seg_start[2]).astype(jnp.int32))

    xf = x.reshape(N, H)
    W_all = jnp.stack([W_sub, W_obj, W_val]).astype(jnp.bfloat16)
    b_all = jnp.stack([b_sub, b_obj, b_val]).reshape(3, 1, H)

    x_sorted = _sc_row_gather(gather_idx, xf, NPAD, chunk=40)

    grid_spec = pltpu.PrefetchScalarGridSpec(
        num_scalar_prefetch=1,
        grid=(NTILES,),
        in_specs=[
            pl.BlockSpec((T, H), lambda t, roles: (t, 0)),
            pl.BlockSpec((1, H, H), lambda t, roles: (roles[t], 0, 0)),
            pl.BlockSpec((1, 1, H), lambda t, roles: (roles[t], 0, 0)),
        ],
        out_specs=pl.BlockSpec((T, H), lambda t, roles: (t, 0)),
    )
    out_sorted = pl.pallas_call(
        _mm_body,
        grid_spec=grid_spec,
        out_shape=jax.ShapeDtypeStruct((NPAD, H), jnp.float32),
    )(tile_role, x_sorted, W_all, b_all)

    out = _sc_row_gather(unsort_idx, out_sorted, N, chunk=32)

    return out.reshape(B, P, HEADS, H // HEADS)


# speedup vs baseline: 1.0450x; 1.0450x over previous
import functools

import jax
import jax.numpy as jnp
from jax import lax
from jax.experimental import pallas as pl
from jax.experimental.pallas import tpu as pltpu
from jax.experimental.pallas import tpu_sc as plsc

HIDDEN = 2048
HEADS = 16
T = 256
NW = 32
CHUNK = 24


def _sc_row_gather(idx, src, n_out):
    H = src.shape[1]
    rows_per_w = n_out // NW
    sizes = [CHUNK] * (rows_per_w // CHUNK)
    if rows_per_w % CHUNK:
        sizes.append(rows_per_w % CHUNK)
    offs = [sum(sizes[:i]) for i in range(len(sizes))]
    mesh = plsc.VectorSubcoreMesh(core_axis_name="c", subcore_axis_name="s")

    @functools.partial(
        pl.kernel,
        out_type=jax.ShapeDtypeStruct((n_out, H), src.dtype),
        mesh=mesh,
        scratch_types=[
            pltpu.VMEM((rows_per_w,), jnp.int32),
            pltpu.VMEM((2, CHUNK, H), src.dtype),
            pltpu.SemaphoreType.DMA,
            pltpu.SemaphoreType.DMA,
        ],
    )
    def gather_k(idx_hbm, src_hbm, out_hbm, idx_v, rows_v, sem0, sem1):
        wid = lax.axis_index("s") * 2 + lax.axis_index("c")
        base = wid * rows_per_w
        pltpu.sync_copy(idx_hbm.at[pl.ds(base, rows_per_w)], idx_v)
        sems = (sem0, sem1)

        def start(c):
            pltpu.make_async_copy(
                src_hbm.at[idx_v.at[pl.ds(offs[c], sizes[c])]],
                rows_v.at[c % 2, pl.ds(0, sizes[c])],
                sems[c % 2],
            ).start()

        def wait(c):
            pltpu.make_async_copy(
                src_hbm.at[idx_v.at[pl.ds(offs[c], sizes[c])]],
                rows_v.at[c % 2, pl.ds(0, sizes[c])],
                sems[c % 2],
            ).wait()

        start(0)
        for c in range(len(sizes)):
            wait(c)
            if c + 1 < len(sizes):
                start(c + 1)
            pltpu.sync_copy(rows_v.at[c % 2, pl.ds(0, sizes[c])],
                            out_hbm.at[pl.ds(base + offs[c], sizes[c])])

    return gather_k(idx, src)


def _mm_body(roles_ref, x_ref, w0_ref, w1_ref, w2_ref, b_ref, out_ref,
             w_vmem, prev_role, sem):
    t = pl.program_id(0)
    role = roles_ref[t]
    w_hbm = (w0_ref, w1_ref, w2_ref)

    @pl.when(jnp.logical_or(t == 0, role != prev_role[0]))
    def _fetch():
        for r in range(3):
            @pl.when(role == r)
            def _():
                pltpu.make_async_copy(w_hbm[r], w_vmem, sem).start()
                pltpu.make_async_copy(w_hbm[r], w_vmem, sem).wait()
        prev_role[0] = role

    acc = lax.dot_general(x_ref[...], w_vmem[...], (((1,), (1,)), ((), ())),
                          preferred_element_type=jnp.float32)
    out_ref[...] = acc + b_ref[0]


def kernel(x, pair_roles, W_sub, b_sub, W_obj, b_obj, W_val, b_val):
    B, P, H = x.shape
    N = B * P
    NPAD = N + 3 * T
    NTILES = NPAD // T

    r32 = pair_roles.astype(jnp.int32)
    roles_flat = jnp.broadcast_to(r32[None, :], (B, P)).reshape(N)
    order = jnp.argsort(roles_flat, stable=True).astype(jnp.int32)
    sorted_roles = roles_flat[order]
    counts = jnp.bincount(roles_flat, length=3)
    seg_len = ((counts + T - 1) // T) * T
    seg_start = jnp.concatenate([jnp.zeros(1, jnp.int32),
                                 jnp.cumsum(seg_len)[:2].astype(jnp.int32)])
    cum_counts = jnp.concatenate([jnp.zeros(1, jnp.int32),
                                  jnp.cumsum(counts)[:2].astype(jnp.int32)])
    j = jnp.arange(N, dtype=jnp.int32)
    padded_pos = seg_start[sorted_roles] + (j - cum_counts[sorted_roles])
    gather_idx = jnp.zeros(NPAD, jnp.int32).at[padded_pos].set(order)
    unsort_idx = jnp.zeros(N, jnp.int32).at[order].set(padded_pos)
    tile_base = jnp.arange(NTILES, dtype=jnp.int32) * T
    tile_role = ((tile_base >= seg_start[1]).astype(jnp.int32)
                 + (tile_base >= seg_start[2]).astype(jnp.int32))

    xf = x.reshape(N, H)
    b_all = jnp.stack([b_sub, b_obj, b_val]).reshape(3, 1, H)

    x_sorted = _sc_row_gather(gather_idx, xf, NPAD)

    grid_spec = pltpu.PrefetchScalarGridSpec(
        num_scalar_prefetch=1,
        grid=(NTILES,),
        in_specs=[
            pl.BlockSpec((T, H), lambda t, roles: (t, 0)),
            pl.BlockSpec(memory_space=pl.ANY),
            pl.BlockSpec(memory_space=pl.ANY),
            pl.BlockSpec(memory_space=pl.ANY),
            pl.BlockSpec((1, 1, H), lambda t, roles: (roles[t], 0, 0)),
        ],
        out_specs=pl.BlockSpec((T, H), lambda t, roles: (t, 0)),
        scratch_shapes=[
            pltpu.VMEM((H, H), jnp.float32),
            pltpu.SMEM((1,), jnp.int32),
            pltpu.SemaphoreType.DMA,
        ],
    )
    out_sorted = pl.pallas_call(
        _mm_body,
        grid_spec=grid_spec,
        out_shape=jax.ShapeDtypeStruct((NPAD, H), jnp.float32),
    )(tile_role, x_sorted, W_sub, W_obj, W_val, b_all)

    out = _sc_row_gather(unsort_idx, out_sorted, N)

    return out.reshape(B, P, HEADS, H // HEADS)


# speedup vs baseline: 1.2384x; 1.1851x over previous
import functools

import jax
import jax.numpy as jnp
from jax import lax
from jax.experimental import pallas as pl
from jax.experimental.pallas import tpu as pltpu
from jax.experimental.pallas import tpu_sc as plsc

HIDDEN = 2048
HEADS = 16
HD = HIDDEN // HEADS
T = 256
NW = 32
CHUNK = 24


def _sc_row_gather(idx, src, n_out):
    row_shape = src.shape[1:]
    rows_per_w = n_out // NW
    sizes = [CHUNK] * (rows_per_w // CHUNK)
    if rows_per_w % CHUNK:
        sizes.append(rows_per_w % CHUNK)
    offs = [sum(sizes[:i]) for i in range(len(sizes))]
    mesh = plsc.VectorSubcoreMesh(core_axis_name="c", subcore_axis_name="s")

    @functools.partial(
        pl.kernel,
        out_type=jax.ShapeDtypeStruct((n_out,) + row_shape, src.dtype),
        mesh=mesh,
        scratch_types=[
            pltpu.VMEM((rows_per_w,), jnp.int32),
            pltpu.VMEM((2, CHUNK) + row_shape, src.dtype),
            pltpu.SemaphoreType.DMA,
            pltpu.SemaphoreType.DMA,
        ],
    )
    def gather_k(idx_hbm, src_hbm, out_hbm, idx_v, rows_v, sem0, sem1):
        wid = lax.axis_index("s") * 2 + lax.axis_index("c")
        base = wid * rows_per_w
        pltpu.sync_copy(idx_hbm.at[pl.ds(base, rows_per_w)], idx_v)
        sems = (sem0, sem1)

        def copy(c):
            return pltpu.make_async_copy(
                src_hbm.at[idx_v.at[pl.ds(offs[c], sizes[c])]],
                rows_v.at[c % 2, pl.ds(0, sizes[c])],
                sems[c % 2],
            )

        copy(0).start()
        for c in range(len(sizes)):
            copy(c).wait()
            if c + 1 < len(sizes):
                copy(c + 1).start()
            pltpu.sync_copy(rows_v.at[c % 2, pl.ds(0, sizes[c])],
                            out_hbm.at[pl.ds(base + offs[c], sizes[c])])

    return gather_k(idx, src)


def _mm_body(roles_ref, x_ref, w0_ref, w1_ref, w2_ref, b_ref, out_ref,
             w_vmem, state, sem):
    t = pl.program_id(0)
    nt = pl.num_programs(0)
    role = roles_ref[t]
    w_hbm = (w0_ref, w1_ref, w2_ref)

    def start_fetch(r, slot):
        for i in range(3):
            @pl.when(r == i)
            def _():
                for s in range(2):
                    @pl.when(slot == s)
                    def _():
                        pltpu.make_async_copy(w_hbm[i], w_vmem.at[s], sem).start()

    def wait_fetch():
        pltpu.make_async_copy(w_hbm[0], w_vmem.at[0], sem).wait()

    @pl.when(t == 0)
    def _first():
        start_fetch(role, 0)
        wait_fetch()
        state[0] = role
        state[1] = 0

    @pl.when(jnp.logical_and(t > 0, role != state[0]))
    def _boundary():
        wait_fetch()
        state[0] = role
        state[1] = 1 - state[1]

    cur_slot = state[1]

    @pl.when(jnp.logical_and(t + 1 < nt, roles_ref[t + 1] != role))
    def _prefetch():
        start_fetch(roles_ref[t + 1], 1 - cur_slot)

    acc = lax.dot_general(x_ref[...], w_vmem[cur_slot],
                          (((1,), (1,)), ((), ())),
                          preferred_element_type=jnp.float32)
    out_ref[...] = (acc + b_ref[0]).reshape(out_ref.shape)


def kernel(x, pair_roles, W_sub, b_sub, W_obj, b_obj, W_val, b_val):
    B, P, H = x.shape
    N = B * P
    NPAD = N + 3 * T
    NTILES = NPAD // T

    r32 = pair_roles.astype(jnp.int32)
    roles_flat = jnp.broadcast_to(r32[None, :], (B, P)).reshape(N)
    onehot = (roles_flat[:, None] == jnp.arange(3, dtype=jnp.int32)[None, :])
    onehot = onehot.astype(jnp.int32)
    incl = jnp.cumsum(onehot, axis=0)
    counts = incl[-1]
    rank = jnp.sum((incl - onehot) * onehot, axis=1)
    seg_len = ((counts + T - 1) // T) * T
    seg_start = jnp.concatenate([jnp.zeros(1, jnp.int32),
                                 jnp.cumsum(seg_len)[:2].astype(jnp.int32)])
    unsort_idx = jnp.sum(seg_start[None, :] * onehot, axis=1) + rank
    gather_idx = jnp.zeros(NPAD, jnp.int32).at[unsort_idx].set(
        jnp.arange(N, dtype=jnp.int32))
    tile_base = jnp.arange(NTILES, dtype=jnp.int32) * T
    tile_role = ((tile_base >= seg_start[1]).astype(jnp.int32)
                 + (tile_base >= seg_start[2]).astype(jnp.int32))

    xf = x.reshape(N, H)
    b_all = jnp.stack([b_sub, b_obj, b_val]).reshape(3, 1, H)

    x_sorted = _sc_row_gather(gather_idx, xf, NPAD)

    grid_spec = pltpu.PrefetchScalarGridSpec(
        num_scalar_prefetch=1,
        grid=(NTILES,),
        in_specs=[
            pl.BlockSpec((T, H), lambda t, roles: (t, 0)),
            pl.BlockSpec(memory_space=pl.ANY),
            pl.BlockSpec(memory_space=pl.ANY),
            pl.BlockSpec(memory_space=pl.ANY),
            pl.BlockSpec((1, 1, H), lambda t, roles: (roles[t], 0, 0)),
        ],
        out_specs=pl.BlockSpec((T, HEADS, HD), lambda t, roles: (t, 0, 0)),
        scratch_shapes=[
            pltpu.VMEM((2, H, H), jnp.float32),
            pltpu.SMEM((2,), jnp.int32),
            pltpu.SemaphoreType.DMA,
        ],
    )
    out_sorted = pl.pallas_call(
        _mm_body,
        grid_spec=grid_spec,
        out_shape=jax.ShapeDtypeStruct((NPAD, HEADS, HD), jnp.float32),
    )(tile_role, x_sorted, W_sub, W_obj, W_val, b_all)

    out = _sc_row_gather(unsort_idx, out_sorted, N)

    return out.reshape(B, P, HEADS, HD)


# speedup vs baseline: 1.2549x; 1.0134x over previous
import functools

import jax
import jax.numpy as jnp
from jax import lax
from jax.experimental import pallas as pl
from jax.experimental.pallas import tpu as pltpu
from jax.experimental.pallas import tpu_sc as plsc

HIDDEN = 2048
HEADS = 16
HD = HIDDEN // HEADS
T = 256
NW = 32
GCHUNK = 24


def _sc_row_gather(idx, src, n_out):
    row_shape = src.shape[1:]
    rows_per_w = n_out // NW
    sizes = [GCHUNK] * (rows_per_w // GCHUNK)
    if rows_per_w % GCHUNK:
        sizes.append(rows_per_w % GCHUNK)
    offs = [sum(sizes[:i]) for i in range(len(sizes))]
    mesh = plsc.VectorSubcoreMesh(core_axis_name="c", subcore_axis_name="s")

    @functools.partial(
        pl.kernel,
        out_type=jax.ShapeDtypeStruct((n_out,) + row_shape, src.dtype),
        mesh=mesh,
        scratch_types=[
            pltpu.VMEM((rows_per_w,), jnp.int32),
            pltpu.VMEM((2, GCHUNK) + row_shape, src.dtype),
            pltpu.SemaphoreType.DMA,
            pltpu.SemaphoreType.DMA,
        ],
    )
    def gather_k(idx_hbm, src_hbm, out_hbm, idx_v, rows_v, sem0, sem1):
        wid = lax.axis_index("s") * 2 + lax.axis_index("c")
        base = wid * rows_per_w
        pltpu.sync_copy(idx_hbm.at[pl.ds(base, rows_per_w)], idx_v)
        sems = (sem0, sem1)

        def copy(c):
            return pltpu.make_async_copy(
                src_hbm.at[idx_v.at[pl.ds(offs[c], sizes[c])]],
                rows_v.at[c % 2, pl.ds(0, sizes[c])],
                sems[c % 2],
            )

        copy(0).start()
        for c in range(len(sizes)):
            copy(c).wait()
            if c + 1 < len(sizes):
                copy(c + 1).start()
            pltpu.sync_copy(rows_v.at[c % 2, pl.ds(0, sizes[c])],
                            out_hbm.at[pl.ds(base + offs[c], sizes[c])])

    return gather_k(idx, src)


def _mm_body(roles_ref, x_ref, w0_ref, w1_ref, w2_ref, b_ref, out_ref,
             w_vmem, state, sem0, sem1, sem2):
    t = pl.program_id(0)
    nt = pl.num_programs(0)
    role = roles_ref[t]
    w_hbm = (w0_ref, w1_ref, w2_ref)
    sems = (sem0, sem1, sem2)

    def wait_w(r):
        for i in range(3):
            @pl.when(r == i)
            def _():
                pltpu.make_async_copy(w_hbm[i], w_vmem.at[i], sems[i]).wait()

    @pl.when(t == 0)
    def _first():
        for i in range(3):
            pltpu.make_async_copy(w_hbm[i], w_vmem.at[i], sems[i]).start()
        wait_w(role)
        state[0] = role
        for i in range(3):
            state[1 + i] = jnp.where(role == i, 1, 0)

    @pl.when(jnp.logical_and(t > 0, role != state[0]))
    def _boundary():
        wait_w(role)
        state[0] = role
        for i in range(3):
            @pl.when(role == i)
            def _():
                state[1 + i] = 1

    acc = lax.dot_general(x_ref[...], w_vmem[state[0]],
                          (((1,), (1,)), ((), ())),
                          preferred_element_type=jnp.float32)
    out_ref[...] = (acc + b_ref[0]).reshape(out_ref.shape)

    @pl.when(t == nt - 1)
    def _drain():
        for i in range(3):
            @pl.when(state[1 + i] == 0)
            def _():
                pltpu.make_async_copy(w_hbm[i], w_vmem.at[i], sems[i]).wait()


def kernel(x, pair_roles, W_sub, b_sub, W_obj, b_obj, W_val, b_val):
    B, P, H = x.shape
    N = B * P
    NPAD = N + 3 * T
    NTILES = NPAD // T

    r32 = pair_roles.astype(jnp.int32)
    roles_flat = jnp.broadcast_to(r32[None, :], (B, P)).reshape(N)
    onehot = (roles_flat[:, None] == jnp.arange(3, dtype=jnp.int32)[None, :])
    onehot = onehot.astype(jnp.int32)
    incl = jnp.cumsum(onehot, axis=0)
    counts = incl[-1]
    rank = jnp.sum((incl - onehot) * onehot, axis=1)
    seg_len = ((counts + T - 1) // T) * T
    seg_start = jnp.concatenate([jnp.zeros(1, jnp.int32),
                                 jnp.cumsum(seg_len)[:2].astype(jnp.int32)])
    unsort_idx = jnp.sum(seg_start[None, :] * onehot, axis=1) + rank
    gather_idx = jnp.zeros(NPAD, jnp.int32).at[unsort_idx].set(
        jnp.arange(N, dtype=jnp.int32))
    tile_base = jnp.arange(NTILES, dtype=jnp.int32) * T
    tile_role = ((tile_base >= seg_start[1]).astype(jnp.int32)
                 + (tile_base >= seg_start[2]).astype(jnp.int32))

    xf = x.reshape(N, H)
    b_all = jnp.stack([b_sub, b_obj, b_val]).reshape(3, 1, H)

    x_sorted = _sc_row_gather(gather_idx, xf, NPAD)

    grid_spec = pltpu.PrefetchScalarGridSpec(
        num_scalar_prefetch=1,
        grid=(NTILES,),
        in_specs=[
            pl.BlockSpec((T, H), lambda t, roles: (t, 0)),
            pl.BlockSpec(memory_space=pl.ANY),
            pl.BlockSpec(memory_space=pl.ANY),
            pl.BlockSpec(memory_space=pl.ANY),
            pl.BlockSpec((1, 1, H), lambda t, roles: (roles[t], 0, 0)),
        ],
        out_specs=pl.BlockSpec((T, HEADS, HD), lambda t, roles: (t, 0, 0)),
        scratch_shapes=[
            pltpu.VMEM((3, H, H), jnp.float32),
            pltpu.SMEM((4,), jnp.int32),
            pltpu.SemaphoreType.DMA,
            pltpu.SemaphoreType.DMA,
            pltpu.SemaphoreType.DMA,
        ],
    )
    out_sorted = pl.pallas_call(
        _mm_body,
        grid_spec=grid_spec,
        out_shape=jax.ShapeDtypeStruct((NPAD, HEADS, HD), jnp.float32),
    )(tile_role, x_sorted, W_sub, W_obj, W_val, b_all)

    out = _sc_row_gather(unsort_idx, out_sorted, N)

    return out.reshape(B, P, HEADS, HD)
